# baseline (speedup 1.0000x reference)
import functools

import jax
import jax.numpy as jnp
from jax import lax
from jax.experimental import pallas as pl
from jax.experimental.pallas import tpu as pltpu
from jax.experimental.pallas import tpu_sc as plsc

TOPK = 5
NROWS = 128
N = 32768
NC = 2
NS = 16
NW = NC * NS
ROWS_PER_W = NROWS // NW
L = 16
U = 8
CH = L * U
NIT = N // CH
CBLK = 2048
NBLK = N // CBLK
F32_MIN = jnp.finfo(jnp.float32).min


def _tc_argmax(y_ref, o_ref, mx_ref, ix_ref):
  j = pl.program_id(0)
  x = y_ref[...]
  cols = lax.broadcasted_iota(jnp.int32, (NROWS, CBLK), 1) + j * CBLK
  m = jnp.max(x, axis=1, keepdims=True)
  idx = jnp.min(jnp.where(x == m, cols, jnp.int32(N)), axis=1, keepdims=True)

  @pl.when(j == 0)
  def _():
    mx_ref[...] = m
    ix_ref[...] = idx

  @pl.when(j > 0)
  def _():
    upd = m > mx_ref[...]
    mx_ref[...] = jnp.where(upd, m, mx_ref[...])
    ix_ref[...] = jnp.where(upd, idx, ix_ref[...])

  @pl.when(j == NBLK - 1)
  def _():
    o_ref[...] = jnp.broadcast_to(jnp.reshape(ix_ref[...], (1, NROWS)),
                                  (8, NROWS))


def _sc_body(yhat_hbm, t_hbm, out_hbm, buf0, buf1, tbuf, obuf,
             sem0, sem1, semt):
  bufs = (buf0, buf1)
  sems = (sem0, sem1)
  wid = lax.axis_index("s") * NC + lax.axis_index("c")
  base_row = wid * ROWS_PER_W
  iota = lax.iota(jnp.int32, L)
  iotas = [iota + u * L for u in range(U)]

  pltpu.make_async_copy(t_hbm, tbuf, semt).start()

  def copy(i):
    return pltpu.make_async_copy(yhat_hbm.at[base_row + i], bufs[i % 2],
                                 sems[i % 2])

  copy(0).start()
  pltpu.make_async_copy(t_hbm, tbuf, semt).wait()

  hits = jnp.float32(0.0)
  for i in range(ROWS_PER_W):
    if i + 1 < ROWS_PER_W:
      copy(i + 1).start()
    copy(i).wait()
    buf = bufs[i % 2]

    r = base_row + i
    tvec = plsc.load_gather(
        tbuf, [jnp.full((L,), 0, jnp.int32),
               jnp.full((L,), r, jnp.int32)])
    t = jnp.max(tvec)
    v = plsc.load_gather(buf, [tvec])
    jb = t // CH

    def pre_body(j, c, buf=buf, v=v):
      base = j * CH
      out = []
      for u in range(U):
        x = buf[pl.ds(base + u * L, L)]
        out.append(c[u] + (x >= v).astype(jnp.int32))
      return tuple(out)

    c = lax.fori_loop(0, jb, pre_body,
                      tuple(jnp.zeros((L,), jnp.int32) for _ in range(U)))

    def suf_body(j, c, buf=buf, v=v):
      base = j * CH
      out = []
      for u in range(U):
        x = buf[pl.ds(base + u * L, L)]
        out.append(c[u] + (x > v).astype(jnp.int32))
      return tuple(out)

    c = lax.fori_loop(jb + 1, NIT, suf_body, c)

    base = jb * CH
    rank = jnp.int32(0)
    for u in range(U):
      x = buf[pl.ds(base + u * L, L)]
      idx = iotas[u] + base
      bc = (x > v) | ((x == v) & (idx < tvec))
      rank = rank + jnp.sum(bc.astype(jnp.int32) + c[u])
    hits = hits + jnp.where(rank < TOPK, jnp.float32(1.0), jnp.float32(0.0))

  obuf[...] = jnp.full((L,), hits, jnp.float32)
  pltpu.sync_copy(obuf, out_hbm.at[wid])


def _tc_merge(p_ref, o_ref):
  total = jnp.sum(p_ref[...]) * (1.0 / L)
  o_ref[...] = jnp.full((1, 1), (1.0 - total / NROWS) * 100.0, jnp.float32)


@jax.jit
def kernel(yhat, y):
  y2d = jnp.reshape(y, (NROWS, N))
  targets = pl.pallas_call(
      _tc_argmax,
      grid=(NBLK,),
      in_specs=[pl.BlockSpec((NROWS, CBLK), lambda j: (0, j))],
      out_specs=pl.BlockSpec((8, NROWS), lambda j: (0, 0)),
      out_shape=jax.ShapeDtypeStruct((8, NROWS), jnp.int32),
      scratch_shapes=[
          pltpu.VMEM((NROWS, 1), jnp.float32),
          pltpu.VMEM((NROWS, 1), jnp.int32),
      ],
  )(y2d)

  mesh = plsc.VectorSubcoreMesh(core_axis_name="c", subcore_axis_name="s")
  sc_k = functools.partial(
      pl.kernel,
      mesh=mesh,
      compiler_params=pltpu.CompilerParams(needs_layout_passes=False,
                                           use_tc_tiling_on_sc=True),
      out_type=jax.ShapeDtypeStruct((NW, L), jnp.float32),
      scratch_types=[
          pltpu.VMEM((N,), jnp.float32),
          pltpu.VMEM((N,), jnp.float32),
          pltpu.VMEM((8, NROWS), jnp.int32),
          pltpu.VMEM((L,), jnp.float32),
          pltpu.SemaphoreType.DMA,
          pltpu.SemaphoreType.DMA,
          pltpu.SemaphoreType.DMA,
      ],
  )(_sc_body)
  partial_hits = sc_k(yhat, targets)

  err = pl.pallas_call(
      _tc_merge,
      out_shape=jax.ShapeDtypeStruct((1, 1), jnp.float32),
  )(partial_hits)
  return jnp.reshape(err, ())
